# baseline (speedup 1.0000x reference)
import jax
import jax.numpy as jnp
from jax import lax
from jax.experimental import pallas as pl
from jax.experimental.pallas import tpu as pltpu
from jax.experimental.pallas import tpu_sc as plsc
import functools

N, E, D, H, G = 10000, 320000, 128, 128, 64
NC, NS = 2, 16
NW = NC * NS
EPW = E // NW
C = 80
NCHUNK = EPW // C
RPS = 624
TAIL = N - NS * RPS

BLK = 2000
NBLK = N // BLK


def _sc_agg_body(x_hbm, src_hbm, dst_hbm, zeros_hbm, out_hbm,
                 s0, s1, s2, s3, d0, d1, d2, d3, r0, r1, r2, r3,
                 i0, i1, i2, i3, g0, g1, g2, g3, agg_sh):
    srcs = [s0, s1, s2, s3]
    dsts = [d0, d1, d2, d3]
    rows = [r0, r1, r2, r3]
    isems = [i0, i1, i2, i3]
    gsems = [g0, g1, g2, g3]
    c = lax.axis_index("c")
    s = lax.axis_index("s")
    wid = c * NS + s

    @pl.when(c == 0)
    def _():
        pltpu.sync_copy(x_hbm.at[pl.ds(s * RPS, RPS)],
                        agg_sh.at[pl.ds(s * RPS, RPS)])

        @pl.when(s == NS - 1)
        def _():
            pltpu.sync_copy(x_hbm.at[pl.ds(NS * RPS, TAIL)],
                            agg_sh.at[pl.ds(NS * RPS, TAIL)])

    @pl.when(c == 1)
    def _():
        pltpu.sync_copy(zeros_hbm, agg_sh.at[pl.ds(s * RPS, RPS)])

        @pl.when(s == NS - 1)
        def _():
            pltpu.sync_copy(zeros_hbm.at[pl.ds(0, TAIL)],
                            agg_sh.at[pl.ds(NS * RPS, TAIL)])

    plsc.subcore_barrier()
    e_base = wid * EPW

    def idx_fetch(j, q):
        base = e_base + j * C
        pltpu.async_copy(src_hbm.at[pl.ds(base, C)], srcs[q], isems[q])
        pltpu.async_copy(dst_hbm.at[pl.ds(base, C)], dsts[q], isems[q])

    def idx_wait(q):
        pltpu.make_async_copy(src_hbm.at[pl.ds(0, C)], srcs[q], isems[q]).wait()
        pltpu.make_async_copy(dst_hbm.at[pl.ds(0, C)], dsts[q], isems[q]).wait()

    def gather(q, k):
        pltpu.async_copy(x_hbm.at[srcs[q]], rows[k], gsems[k])

    def gather_wait(q, k):
        pltpu.make_async_copy(x_hbm.at[srcs[q]], rows[k], gsems[k]).wait()

    def scat(k, q):
        pltpu.sync_copy(rows[k], agg_sh.at[dsts[q]], add=True)

    def phase(j, m, fetch, issue):
        m3 = (m + 3) % 4
        if issue:
            idx_wait(m3)
        gather_wait(m, m)
        if issue:
            gather(m3, m3)
        scat(m, m)
        if fetch:
            idx_fetch(j + 4, m)

    for q in range(4):
        idx_fetch(q, q)
    for q in range(3):
        idx_wait(q)
        gather(q, q)

    def body(i, carry):
        for m in range(4):
            phase(4 * i + m, m, True, True)
        return carry

    lax.fori_loop(0, 30, body, 0)
    phase(120, 0, True, True)
    phase(121, 1, False, True)
    phase(122, 2, False, False)
    phase(123, 3, False, False)
    phase(124, 0, False, False)
    plsc.subcore_barrier()

    pltpu.sync_copy(agg_sh.at[pl.ds(s * RPS, RPS)],
                    out_hbm.at[c, pl.ds(s * RPS, RPS)])

    @pl.when(s == NS - 1)
    def _():
        pltpu.sync_copy(agg_sh.at[pl.ds(NS * RPS, TAIL)],
                        out_hbm.at[c, pl.ds(NS * RPS, TAIL)])


@functools.cache
def _sc_agg():
    return pl.kernel(
        _sc_agg_body,
        out_type=jax.ShapeDtypeStruct((NC, N, D), jnp.float32),
        mesh=plsc.VectorSubcoreMesh(core_axis_name="c", subcore_axis_name="s",
                                    num_cores=NC, num_subcores=NS),
        scratch_types=(
            [pltpu.VMEM((C,), jnp.int32)] * 8
            + [pltpu.VMEM((C, D), jnp.float32)] * 4
            + [pltpu.SemaphoreType.DMA] * 8
            + [pltpu.VMEM_SHARED((N, D), jnp.float32)]
        ),
    )


def _tc_body(parts_ref, batch_ref, W1_ref, b1_ref, W2_ref, b2_ref,
             W3_ref, b3_ref, out_ref, pooled_acc):
    i = pl.program_id(0)
    h = parts_ref[0] + parts_ref[1]
    h1 = jnp.dot(h, W1_ref[...], preferred_element_type=jnp.float32)
    h1 = jnp.maximum(h1 + b1_ref[...], 0.0)
    h2 = jnp.dot(h1, W2_ref[...], preferred_element_type=jnp.float32)
    h2 = h2 + b2_ref[...]
    bm = batch_ref[0, 0, :]
    gids = lax.broadcasted_iota(jnp.int32, (G, BLK), 0)
    mask = (bm[None, :] == gids).astype(jnp.float32)
    p = jnp.dot(mask, h2, preferred_element_type=jnp.float32)

    @pl.when(i == 0)
    def _():
        pooled_acc[...] = jnp.zeros_like(pooled_acc)

    pooled_acc[...] += p

    @pl.when(i == pl.num_programs(0) - 1)
    def _():
        out_ref[...] = (jnp.dot(pooled_acc[...], W3_ref[...],
                                preferred_element_type=jnp.float32)
                        + b3_ref[...])


@functools.partial(jax.jit)
def _tc_mlp_pool(parts, batch3, W1, b1, W2, b2, W3, b3):
    return pl.pallas_call(
        _tc_body,
        grid=(NBLK,),
        in_specs=[
            pl.BlockSpec((NC, BLK, D), lambda i: (0, i, 0)),
            pl.BlockSpec((1, 1, BLK), lambda i: (i, 0, 0)),
            pl.BlockSpec((D, H), lambda i: (0, 0)),
            pl.BlockSpec((1, H), lambda i: (0, 0)),
            pl.BlockSpec((H, H), lambda i: (0, 0)),
            pl.BlockSpec((1, H), lambda i: (0, 0)),
            pl.BlockSpec((H, 1), lambda i: (0, 0)),
            pl.BlockSpec((1, 1), lambda i: (0, 0)),
        ],
        out_specs=pl.BlockSpec((G, 1), lambda i: (0, 0)),
        out_shape=jax.ShapeDtypeStruct((G, 1), jnp.float32),
        scratch_shapes=[pltpu.VMEM((G, H), jnp.float32)],
        compiler_params=pltpu.CompilerParams(
            dimension_semantics=("arbitrary",)),
    )(parts, batch3, W1, b1, W2, b2, W3, b3)


def kernel(x, edge_index, batch, W1, b1, W2, b2, W3, b3):
    src = edge_index[0]
    dst = edge_index[1]
    zeros = jnp.zeros((RPS, D), x.dtype)
    parts = _sc_agg()(x, src, dst, zeros)
    out = _tc_mlp_pool(parts, batch.reshape(NBLK, 1, BLK),
                       W1, b1.reshape(1, H), W2, b2.reshape(1, H),
                       W3, b3.reshape(1, 1))
    return out


# speedup vs baseline: 1.2306x; 1.2306x over previous
import jax
import jax.numpy as jnp
from jax import lax
from jax.experimental import pallas as pl
from jax.experimental.pallas import tpu as pltpu
from jax.experimental.pallas import tpu_sc as plsc
import functools

N, E, D, H, G = 10000, 320000, 128, 128, 64
NC, NS = 2, 16
NW = NC * NS
EPW = E // NW
C = 80
NCHUNK = EPW // C
RPS = 624
TAIL = N - NS * RPS

BLK = 2000
NBLK = N // BLK


def _sc_agg_body(x_hbm, src_hbm, dst_hbm, zeros_hbm, out_hbm,
                 s0, s1, s2, s3, s4, s5, s6, s7,
                 d0, d1, d2, d3, d4, d5, d6, d7,
                 r0, r1, r2, r3,
                 i0, i1, i2, i3, i4, i5, i6, i7,
                 g0, g1, g2, g3, t0, t1, t2, t3, agg_sh):
    srcs = [s0, s1, s2, s3, s4, s5, s6, s7]
    dsts = [d0, d1, d2, d3, d4, d5, d6, d7]
    rows = [r0, r1, r2, r3]
    isems = [i0, i1, i2, i3, i4, i5, i6, i7]
    gsems = [g0, g1, g2, g3]
    ssems = [t0, t1, t2, t3]
    c = lax.axis_index("c")
    s = lax.axis_index("s")
    wid = c * NS + s

    @pl.when(c == 0)
    def _():
        pltpu.sync_copy(x_hbm.at[pl.ds(s * RPS, RPS)],
                        agg_sh.at[pl.ds(s * RPS, RPS)])

        @pl.when(s == NS - 1)
        def _():
            pltpu.sync_copy(x_hbm.at[pl.ds(NS * RPS, TAIL)],
                            agg_sh.at[pl.ds(NS * RPS, TAIL)])

    @pl.when(c == 1)
    def _():
        pltpu.sync_copy(zeros_hbm, agg_sh.at[pl.ds(s * RPS, RPS)])

        @pl.when(s == NS - 1)
        def _():
            pltpu.sync_copy(zeros_hbm.at[pl.ds(0, TAIL)],
                            agg_sh.at[pl.ds(NS * RPS, TAIL)])

    plsc.subcore_barrier()
    e_base = wid * EPW

    def idx_fetch(j, q):
        base = e_base + j * C
        pltpu.async_copy(src_hbm.at[pl.ds(base, C)], srcs[q], isems[q])
        pltpu.async_copy(dst_hbm.at[pl.ds(base, C)], dsts[q], isems[q])

    def idx_wait(q):
        pltpu.make_async_copy(src_hbm.at[pl.ds(0, C)], srcs[q], isems[q]).wait()
        pltpu.make_async_copy(dst_hbm.at[pl.ds(0, C)], dsts[q], isems[q]).wait()

    def gather(q, k):
        pltpu.async_copy(x_hbm.at[srcs[q]], rows[k], gsems[k])

    def gather_wait(q, k):
        pltpu.make_async_copy(x_hbm.at[srcs[q]], rows[k], gsems[k]).wait()

    def scat_start(k, q):
        pltpu.async_copy(rows[k], agg_sh.at[dsts[q]], ssems[k], add=True)

    def scat_wait(k, q):
        pltpu.make_async_copy(rows[k], agg_sh.at[dsts[q]], ssems[k]).wait()

    def phase(j, m, q, fetch, issue, swait):
        m3, q3, qp = (m + 3) % 4, (q + 3) % 8, (q + 7) % 8
        if issue:
            idx_wait(q3)
        gather_wait(q, m)
        scat_start(m, q)
        if swait:
            scat_wait(m3, qp)
        if issue:
            gather(q3, m3)
        if fetch:
            idx_fetch(j + 7, qp)

    for q in range(7):
        idx_fetch(q, q)
    for q in range(3):
        idx_wait(q)
        gather(q, q)
    for j in range(8):
        phase(j, j % 4, j % 8, True, True, j > 0)

    def body(i, carry):
        jb = 8 * i + 8
        for u in range(8):
            phase(jb + u, u % 4, u % 8, True, True, True)
        return carry

    lax.fori_loop(0, 13, body, 0)
    for j in range(112, 125):
        phase(j, j % 4, j % 8, j <= 117, j <= 121, True)
    scat_wait(0, 4)
    plsc.subcore_barrier()

    pltpu.sync_copy(agg_sh.at[pl.ds(s * RPS, RPS)],
                    out_hbm.at[c, pl.ds(s * RPS, RPS)])

    @pl.when(s == NS - 1)
    def _():
        pltpu.sync_copy(agg_sh.at[pl.ds(NS * RPS, TAIL)],
                        out_hbm.at[c, pl.ds(NS * RPS, TAIL)])


@functools.cache
def _sc_agg():
    return pl.kernel(
        _sc_agg_body,
        out_type=jax.ShapeDtypeStruct((NC, N, D), jnp.float32),
        mesh=plsc.VectorSubcoreMesh(core_axis_name="c", subcore_axis_name="s",
                                    num_cores=NC, num_subcores=NS),
        scratch_types=(
            [pltpu.VMEM((C,), jnp.int32)] * 16
            + [pltpu.VMEM((C, D), jnp.float32)] * 4
            + [pltpu.SemaphoreType.DMA] * 16
            + [pltpu.VMEM_SHARED((N, D), jnp.float32)]
        ),
    )


def _tc_body(parts_ref, batch_ref, W1_ref, b1_ref, W2_ref, b2_ref,
             W3_ref, b3_ref, out_ref, pooled_acc):
    i = pl.program_id(0)
    h = parts_ref[0] + parts_ref[1]
    h1 = jnp.dot(h, W1_ref[...], preferred_element_type=jnp.float32)
    h1 = jnp.maximum(h1 + b1_ref[...], 0.0)
    h2 = jnp.dot(h1, W2_ref[...], preferred_element_type=jnp.float32)
    h2 = h2 + b2_ref[...]
    bm = batch_ref[0, 0, :]
    gids = lax.broadcasted_iota(jnp.int32, (G, BLK), 0)
    mask = (bm[None, :] == gids).astype(jnp.float32)
    p = jnp.dot(mask, h2, preferred_element_type=jnp.float32)

    @pl.when(i == 0)
    def _():
        pooled_acc[...] = jnp.zeros_like(pooled_acc)

    pooled_acc[...] += p

    @pl.when(i == pl.num_programs(0) - 1)
    def _():
        out_ref[...] = (jnp.dot(pooled_acc[...], W3_ref[...],
                                preferred_element_type=jnp.float32)
                        + b3_ref[...])


@functools.partial(jax.jit)
def _tc_mlp_pool(parts, batch3, W1, b1, W2, b2, W3, b3):
    return pl.pallas_call(
        _tc_body,
        grid=(NBLK,),
        in_specs=[
            pl.BlockSpec((NC, BLK, D), lambda i: (0, i, 0)),
            pl.BlockSpec((1, 1, BLK), lambda i: (i, 0, 0)),
            pl.BlockSpec((D, H), lambda i: (0, 0)),
            pl.BlockSpec((1, H), lambda i: (0, 0)),
            pl.BlockSpec((H, H), lambda i: (0, 0)),
            pl.BlockSpec((1, H), lambda i: (0, 0)),
            pl.BlockSpec((H, 1), lambda i: (0, 0)),
            pl.BlockSpec((1, 1), lambda i: (0, 0)),
        ],
        out_specs=pl.BlockSpec((G, 1), lambda i: (0, 0)),
        out_shape=jax.ShapeDtypeStruct((G, 1), jnp.float32),
        scratch_shapes=[pltpu.VMEM((G, H), jnp.float32)],
        compiler_params=pltpu.CompilerParams(
            dimension_semantics=("arbitrary",)),
    )(parts, batch3, W1, b1, W2, b2, W3, b3)


def kernel(x, edge_index, batch, W1, b1, W2, b2, W3, b3):
    src = edge_index[0]
    dst = edge_index[1]
    zeros = jnp.zeros((RPS, D), x.dtype)
    parts = _sc_agg()(x, src, dst, zeros)
    out = _tc_mlp_pool(parts, batch.reshape(NBLK, 1, BLK),
                       W1, b1.reshape(1, H), W2, b2.reshape(1, H),
                       W3, b3.reshape(1, 1))
    return out
